# baseline (speedup 1.0000x reference)
import functools

import jax
import jax.numpy as jnp
from jax import lax
from jax.experimental import pallas as pl
from jax.experimental.pallas import tpu as pltpu
from jax.experimental.pallas import tpu_sc as plsc

_B, _C, _D = 128, 1024, 256
_NC, _NS = 2, 16
_NW = _NC * _NS
_BPW = _B // _NW
_CHUNK = 64
_CPB = _C // _CHUNK
_NBUF = 4
_T = _BPW * _CPB
_NGRP = _T // _NBUF


def _worker_body(x_hbm, idx_hbm, out_hbm, idx_v, rows_v, gsems, wsems):
    wid = lax.axis_index("s") * _NC + lax.axis_index("c")
    pltpu.sync_copy(idx_hbm, idx_v)
    b0 = wid * _BPW

    def gather(t, slot):
        b = b0 + t // _CPB
        koff = pl.multiple_of((t % _CPB) * _CHUNK, _CHUNK)
        pltpu.async_copy(
            x_hbm.at[b].at[idx_v.at[pl.ds(koff, _CHUNK)]],
            rows_v.at[slot],
            gsems[slot],
        )

    def wait_gather(slot):
        pltpu.make_async_copy(
            x_hbm.at[0].at[idx_v.at[pl.ds(0, _CHUNK)]],
            rows_v.at[slot],
            gsems[slot],
        ).wait()

    def write_back(t, slot):
        b = b0 + t // _CPB
        koff = pl.multiple_of((t % _CPB) * _CHUNK, _CHUNK)
        pltpu.async_copy(
            rows_v.at[slot],
            out_hbm.at[b].at[pl.ds(koff, _CHUNK)],
            wsems[slot],
        )

    def wait_write(slot):
        pltpu.make_async_copy(
            rows_v.at[slot],
            out_hbm.at[0].at[pl.ds(0, _CHUNK)],
            wsems[slot],
        ).wait()

    def group_body(g, carry):
        t0 = g * _NBUF
        for s in range(_NBUF):

            @pl.when(g > 0)
            def _():
                wait_write(s)

            gather(t0 + s, s)
            prev = (s - 1) % _NBUF
            if s == 0:

                @pl.when(g > 0)
                def _():
                    wait_gather(prev)
                    write_back(t0 - 1, prev)

            else:
                wait_gather(prev)
                write_back(t0 + s - 1, prev)
        return carry

    lax.fori_loop(0, _NGRP, group_body, 0)
    last = _NBUF - 1
    wait_gather(last)
    write_back(_T - 1, last)
    for s in range(_NBUF):
        wait_write(s)


@jax.jit
def _sc_subsample(x, idx):
    mesh = plsc.VectorSubcoreMesh(core_axis_name="c", subcore_axis_name="s")
    f = pl.kernel(
        _worker_body,
        mesh=mesh,
        out_type=jax.ShapeDtypeStruct((_B, _C, _D), jnp.float32),
        compiler_params=pltpu.CompilerParams(use_tc_tiling_on_sc=False),
        scratch_types=[
            pltpu.VMEM((_C,), jnp.int32),
            pltpu.VMEM((_NBUF, _CHUNK, _D), jnp.float32),
            [pltpu.SemaphoreType.DMA] * _NBUF,
            [pltpu.SemaphoreType.DMA] * _NBUF,
        ],
    )
    return f(x, idx)


def kernel(x, forward_shuffle_idx):
    return _sc_subsample(x, forward_shuffle_idx)


# speedup vs baseline: 3.1375x; 3.1375x over previous
import functools

import jax
import jax.numpy as jnp
from jax import lax
from jax.experimental import pallas as pl
from jax.experimental.pallas import tpu as pltpu
from jax.experimental.pallas import tpu_sc as plsc

_B, _C, _D = 128, 1024, 256
_NC, _NS = 2, 16
_NW = _NC * _NS
_BPW = _B // _NW
_CHUNK = 64
_CPB = _C // _CHUNK
_NBUF = 4
_T = _BPW * _CPB
_NGRP = _T // _NBUF


def _worker_body(x_hbm, idx_hbm, out_hbm, idx_v, rows_v, gsems, wsems):
    wid = lax.axis_index("s") * _NC + lax.axis_index("c")
    pltpu.sync_copy(idx_hbm, idx_v)
    def gather(t, slot):
        b = wid + (t // _CPB) * _NW
        koff = pl.multiple_of((t % _CPB) * _CHUNK, _CHUNK)
        pltpu.async_copy(
            x_hbm.at[b].at[idx_v.at[pl.ds(koff, _CHUNK)]],
            rows_v.at[slot],
            gsems[slot],
        )

    def wait_gather(slot):
        pltpu.make_async_copy(
            x_hbm.at[0].at[idx_v.at[pl.ds(0, _CHUNK)]],
            rows_v.at[slot],
            gsems[slot],
        ).wait()

    def write_back(t, slot):
        b = wid + (t // _CPB) * _NW
        koff = pl.multiple_of((t % _CPB) * _CHUNK, _CHUNK)
        pltpu.async_copy(
            rows_v.at[slot],
            out_hbm.at[b].at[pl.ds(koff, _CHUNK)],
            wsems[slot],
        )

    def wait_write(slot):
        pltpu.make_async_copy(
            rows_v.at[slot],
            out_hbm.at[0].at[pl.ds(0, _CHUNK)],
            wsems[slot],
        ).wait()

    def group_body(g, carry):
        t0 = g * _NBUF
        for s in range(_NBUF):

            @pl.when(g > 0)
            def _():
                wait_write(s)

            gather(t0 + s, s)
            prev = (s - 1) % _NBUF
            if s == 0:

                @pl.when(g > 0)
                def _():
                    wait_gather(prev)
                    write_back(t0 - 1, prev)

            else:
                wait_gather(prev)
                write_back(t0 + s - 1, prev)
        return carry

    lax.fori_loop(0, _NGRP, group_body, 0)
    last = _NBUF - 1
    wait_gather(last)
    write_back(_T - 1, last)
    for s in range(_NBUF):
        wait_write(s)


@jax.jit
def _sc_subsample(x, idx):
    mesh = plsc.VectorSubcoreMesh(core_axis_name="c", subcore_axis_name="s")
    f = pl.kernel(
        _worker_body,
        mesh=mesh,
        out_type=jax.ShapeDtypeStruct((_B, _C, _D), jnp.float32),
        scratch_types=[
            pltpu.VMEM((_C,), jnp.int32),
            pltpu.VMEM((_NBUF, _CHUNK, _D), jnp.float32),
            [pltpu.SemaphoreType.DMA] * _NBUF,
            [pltpu.SemaphoreType.DMA] * _NBUF,
        ],
    )
    return f(x, idx)


def kernel(x, forward_shuffle_idx):
    return _sc_subsample(x, forward_shuffle_idx)
